# baseline (speedup 1.0000x reference)
import functools

import jax
import jax.numpy as jnp
from jax import lax
from jax.experimental import pallas as pl
from jax.experimental.pallas import tpu as pltpu
from jax.experimental.pallas import tpu_sc as plsc

B = 16384
D = 64
NW = 32
BPW = B // NW
CHUNK = 128
NCH = BPW // CHUNK
L = 16


def _body(h_hbm, r_hbm, t_hbm, ent_hbm, rel_hbm, out_hbm,
          idx_h, idx_r, idx_t, rows_h, rows_r, rows_t, out_v, sem):
    cid = lax.axis_index("c")
    sid = lax.axis_index("s")
    wid = sid * 2 + cid
    base = wid * BPW

    for j in range(NCH):
        sl = pl.ds(base + j * CHUNK, CHUNK)
        pltpu.sync_copy(h_hbm.at[sl], idx_h.at[j])
        pltpu.sync_copy(r_hbm.at[sl], idx_r.at[j])
        pltpu.sync_copy(t_hbm.at[sl], idx_t.at[j])

    copies = []
    for j in range(NCH):
        dst = pl.ds(j * CHUNK, CHUNK)
        copies.append(pltpu.async_copy(ent_hbm.at[idx_h.at[j]], rows_h.at[dst], sem))
        copies.append(pltpu.async_copy(rel_hbm.at[idx_r.at[j]], rows_r.at[dst], sem))
        copies.append(pltpu.async_copy(ent_hbm.at[idx_t.at[j]], rows_t.at[dst], sem))
    for c in copies:
        c.wait()

    lane = lax.iota(jnp.int32, L)

    def group(g, _):
        base_row = g * L
        res = jnp.zeros((L,), jnp.float32)
        for rloc in range(L):
            i = base_row + rloc
            acc = jnp.abs(rows_h[i, pl.ds(0, L)] + rows_r[i, pl.ds(0, L)]
                          - rows_t[i, pl.ds(0, L)])
            for c in range(1, D // L):
                sl = pl.ds(c * L, L)
                acc = acc + jnp.abs(rows_h[i, sl] + rows_r[i, sl] - rows_t[i, sl])
            res = jnp.where(lane == rloc, jnp.sum(acc), res)
        out_v[pl.ds(base_row, L)] = -res
        return 0

    lax.fori_loop(0, BPW // L, group, 0)

    pltpu.sync_copy(out_v, out_hbm.at[pl.ds(base, BPW)])


@jax.jit
def kernel(h, r, t, entity_table, relation_table):
    k = pl.kernel(
        _body,
        mesh=plsc.VectorSubcoreMesh(core_axis_name="c", subcore_axis_name="s"),
        out_type=jax.ShapeDtypeStruct((B,), jnp.float32),
        compiler_params=pltpu.CompilerParams(
            needs_layout_passes=False, use_tc_tiling_on_sc=False),
        scratch_types=[
            pltpu.VMEM((NCH, CHUNK), jnp.int32),
            pltpu.VMEM((NCH, CHUNK), jnp.int32),
            pltpu.VMEM((NCH, CHUNK), jnp.int32),
            pltpu.VMEM((BPW, D), jnp.float32),
            pltpu.VMEM((BPW, D), jnp.float32),
            pltpu.VMEM((BPW, D), jnp.float32),
            pltpu.VMEM((BPW,), jnp.float32),
            pltpu.SemaphoreType.DMA,
        ],
    )
    return k(h, r, t, entity_table, relation_table)
